# baseline (speedup 1.0000x reference)
import jax
import jax.numpy as jnp
from jax.experimental import pallas as pl
from jax.experimental.pallas import tpu as pltpu


def _body(idx_ref, emb_ref, w_ref, b_ref, buf_hbm, out_hbm, proj_vmem, sem0, sem1):
    cp = pltpu.make_async_copy(buf_hbm, out_hbm, sem0)
    cp.start()
    proj_vmem[...] = (
        jnp.dot(emb_ref[...], w_ref[...], preferred_element_type=jnp.float32)
        + b_ref[...]
    )
    cp.wait()
    idx = idx_ref[0]
    rcp = pltpu.make_async_copy(proj_vmem, out_hbm.at[pl.ds(idx, 1), :], sem1)
    rcp.start()
    rcp.wait()


def kernel(embedding, buffer, pointer, W, b):
    max_steps, hidden = buffer.shape
    if embedding.ndim == 1:
        embedding = embedding[None, :]
    idx = (jnp.asarray(pointer, jnp.int32) % max_steps).reshape((1,))
    b2 = b.reshape(1, hidden)

    grid_spec = pltpu.PrefetchScalarGridSpec(
        num_scalar_prefetch=1,
        grid=(1,),
        in_specs=[
            pl.BlockSpec((1, hidden), lambda i, idx_ref: (0, 0)),
            pl.BlockSpec((hidden, hidden), lambda i, idx_ref: (0, 0)),
            pl.BlockSpec((1, hidden), lambda i, idx_ref: (0, 0)),
            pl.BlockSpec(memory_space=pltpu.MemorySpace.HBM),
        ],
        out_specs=pl.BlockSpec(memory_space=pltpu.MemorySpace.HBM),
        scratch_shapes=[
            pltpu.VMEM((1, hidden), jnp.float32),
            pltpu.SemaphoreType.DMA,
            pltpu.SemaphoreType.DMA,
        ],
    )
    return pl.pallas_call(
        _body,
        grid_spec=grid_spec,
        out_shape=jax.ShapeDtypeStruct((max_steps, hidden), jnp.float32),
    )(idx, embedding, W, b2, buffer)


# speedup vs baseline: 1.0006x; 1.0006x over previous
import jax
import jax.numpy as jnp
from jax.experimental import pallas as pl
from jax.experimental.pallas import tpu as pltpu

NCHUNK = 16
ROWS = 6250


def _body(idx_ref, emb_ref, w_ref, b_ref, buf_hbm, out_hbm, proj_vmem, sems, sem1):
    for c in range(NCHUNK):
        pltpu.make_async_copy(
            buf_hbm.at[pl.ds(c * ROWS, ROWS), :],
            out_hbm.at[pl.ds(c * ROWS, ROWS), :],
            sems.at[c],
        ).start()
    proj_vmem[...] = (
        jnp.dot(emb_ref[...], w_ref[...], preferred_element_type=jnp.float32)
        + b_ref[...]
    )
    for c in range(NCHUNK):
        pltpu.make_async_copy(
            buf_hbm.at[pl.ds(c * ROWS, ROWS), :],
            out_hbm.at[pl.ds(c * ROWS, ROWS), :],
            sems.at[c],
        ).wait()
    idx = idx_ref[0]
    rcp = pltpu.make_async_copy(proj_vmem, out_hbm.at[pl.ds(idx, 1), :], sem1)
    rcp.start()
    rcp.wait()


def kernel(embedding, buffer, pointer, W, b):
    max_steps, hidden = buffer.shape
    if embedding.ndim == 1:
        embedding = embedding[None, :]
    idx = (jnp.asarray(pointer, jnp.int32) % max_steps).reshape((1,))
    b2 = b.reshape(1, hidden)

    grid_spec = pltpu.PrefetchScalarGridSpec(
        num_scalar_prefetch=1,
        grid=(1,),
        in_specs=[
            pl.BlockSpec((1, hidden), lambda i, idx_ref: (0, 0)),
            pl.BlockSpec((hidden, hidden), lambda i, idx_ref: (0, 0)),
            pl.BlockSpec((1, hidden), lambda i, idx_ref: (0, 0)),
            pl.BlockSpec(memory_space=pltpu.MemorySpace.HBM),
        ],
        out_specs=pl.BlockSpec(memory_space=pltpu.MemorySpace.HBM),
        scratch_shapes=[
            pltpu.VMEM((1, hidden), jnp.float32),
            pltpu.SemaphoreType.DMA((NCHUNK,)),
            pltpu.SemaphoreType.DMA,
        ],
    )
    return pl.pallas_call(
        _body,
        grid_spec=grid_spec,
        out_shape=jax.ShapeDtypeStruct((max_steps, hidden), jnp.float32),
    )(idx, embedding, W, b2, buffer)


# speedup vs baseline: 48.7422x; 48.7113x over previous
import jax
import jax.numpy as jnp
from jax.experimental import pallas as pl
from jax.experimental.pallas import tpu as pltpu

BLOCK = 20000


def _body(idx_ref, emb_ref, w_ref, b_ref, buf_ref, out_ref):
    out_ref[...] = buf_ref[...]
    i = pl.program_id(0)
    idx = idx_ref[0]
    blk = idx // BLOCK

    @pl.when(i == blk)
    def _():
        proj = (
            jnp.dot(emb_ref[...], w_ref[...], preferred_element_type=jnp.float32)
            + b_ref[...]
        )
        row = idx - blk * BLOCK
        out_ref[pl.ds(row, 1), :] = proj


def kernel(embedding, buffer, pointer, W, b):
    max_steps, hidden = buffer.shape
    if embedding.ndim == 1:
        embedding = embedding[None, :]
    idx = (jnp.asarray(pointer, jnp.int32) % max_steps).reshape((1,))
    b2 = b.reshape(1, hidden)
    n_blocks = max_steps // BLOCK

    grid_spec = pltpu.PrefetchScalarGridSpec(
        num_scalar_prefetch=1,
        grid=(n_blocks,),
        in_specs=[
            pl.BlockSpec((1, hidden), lambda i, idx_ref: (0, 0)),
            pl.BlockSpec((hidden, hidden), lambda i, idx_ref: (0, 0)),
            pl.BlockSpec((1, hidden), lambda i, idx_ref: (0, 0)),
            pl.BlockSpec((BLOCK, hidden), lambda i, idx_ref: (i, 0)),
        ],
        out_specs=pl.BlockSpec((BLOCK, hidden), lambda i, idx_ref: (i, 0)),
    )
    return pl.pallas_call(
        _body,
        grid_spec=grid_spec,
        out_shape=jax.ShapeDtypeStruct((max_steps, hidden), jnp.float32),
    )(idx, embedding, W, b2, buffer)
